# baseline (speedup 1.0000x reference)
import functools

import jax
import jax.numpy as jnp
from jax import lax
from jax.experimental import pallas as pl
from jax.experimental.pallas import tpu as pltpu
from jax.experimental.pallas import tpu_sc as plsc

_C = 16
_N = 2_000_000
_NW = 32
_L = 16
_B = 256
_NFB = 244
_NBLK = _NFB + 1
_PADP = _NW * _NBLK * _B
_V = 128 * 128 * 128
_FULL_PER_W = _NFB * _B


def _sc_sampler():
    mesh = plsc.VectorSubcoreMesh(core_axis_name="c", subcore_axis_name="s")

    @functools.partial(
        pl.kernel,
        mesh=mesh,
        compiler_params=pltpu.CompilerParams(use_tc_tiling_on_sc=False),
        out_type=jax.ShapeDtypeStruct((_N, _C), jnp.float32),
        scratch_types=[
            pltpu.VMEM((3 * _B,), jnp.float32),
            pltpu.VMEM((3 * _B,), jnp.float32),
            pltpu.VMEM((8 * _B,), jnp.int32),
            pltpu.VMEM((8 * _B,), jnp.int32),
            pltpu.VMEM((3 * _B,), jnp.float32),
            pltpu.VMEM((3 * _B,), jnp.float32),
            pltpu.VMEM((8 * _B, _C), jnp.float32),
            pltpu.VMEM((8 * _B, _C), jnp.float32),
            pltpu.VMEM((_B, _C), jnp.float32),
            pltpu.SemaphoreType.DMA,
            pltpu.SemaphoreType.DMA,
        ],
    )
    def sampler(tab_hbm, crd_hbm, out_hbm, cb0, cb1, ix0, ix1, wb0, wb1,
                rw0, rw1, out_v, sg0, sg1):
        wid = lax.axis_index("s") * 2 + lax.axis_index("c")
        gb_base = wid * _NBLK
        obase = wid * _FULL_PER_W + 32 * jnp.minimum(wid, 24) \
            + 48 * jnp.maximum(wid - 24, 0)
        tailg = jnp.where(wid < 24, 2, 3)
        cbufs, ixs, wbs, rws, sgs = (cb0, cb1), (ix0, ix1), (wb0, wb1), \
            (rw0, rw1), (sg0, sg1)

        def prep(gb, par):
            cbuf, ixb, wbf, rwb, sem = \
                cbufs[par], ixs[par], wbs[par], rws[par], sgs[par]
            pltpu.sync_copy(crd_hbm.at[pl.ds(gb * 3 * _B, 3 * _B)], cbuf)

            def phase_a(g, c2):
                s = g * _L
                fx = jnp.clip(cbuf[pl.ds(s, _L)] * 63.5 + 63.5, 0.0, 127.0)
                fy = jnp.clip(cbuf[pl.ds(_B + s, _L)] * 63.5 + 63.5,
                              0.0, 127.0)
                fz = jnp.clip(cbuf[pl.ds(2 * _B + s, _L)] * 63.5 + 63.5,
                              0.0, 127.0)
                x0 = fx.astype(jnp.int32)
                y0 = fy.astype(jnp.int32)
                z0 = fz.astype(jnp.int32)
                wx1 = fx - x0.astype(jnp.float32)
                wy1 = fy - y0.astype(jnp.float32)
                wz1 = fz - z0.astype(jnp.float32)
                x1 = jnp.minimum(x0 + 1, 127)
                ys0 = y0 << 7
                ys1 = jnp.minimum(y0 + 1, 127) << 7
                zs0 = z0 << 14
                zs1 = jnp.minimum(z0 + 1, 127) << 14
                b00 = zs0 + ys0
                b01 = zs0 + ys1
                b10 = zs1 + ys0
                b11 = zs1 + ys1
                bases = (b00, b00, b01, b01, b10, b10, b11, b11)
                xks = (x0, x1, x0, x1, x0, x1, x0, x1)
                for k in range(8):
                    ixb[pl.ds(k * _B + s, _L)] = bases[k] + xks[k]
                wbf[pl.ds(s, _L)] = wx1
                wbf[pl.ds(_B + s, _L)] = wy1
                wbf[pl.ds(2 * _B + s, _L)] = wz1
                return c2

            lax.fori_loop(0, _B // _L, phase_a, 0)
            for j in range(8 * (_B // 128)):
                jsl = pl.ds(j * 128, 128)
                pltpu.async_copy(tab_hbm.at[ixb.at[jsl]], rwb.at[jsl], sem)

        def drain(par):
            pltpu.make_async_copy(tab_hbm.at[pl.ds(0, 8 * _B)], rws[par],
                                  sgs[par]).wait()

        def combine_group(g, par):
            wbf, rwb = wbs[par], rws[par]
            s = g * _L
            wxv = wbf[pl.ds(s, _L)]
            wyv = wbf[pl.ds(_B + s, _L)]
            wzv = wbf[pl.ds(2 * _B + s, _L)]
            for p in range(_L):
                pt = s + p
                wx, wy, wz = wxv[p], wyv[p], wzv[p]
                c000 = rwb[pt]
                c001 = rwb[_B + pt]
                c010 = rwb[2 * _B + pt]
                c011 = rwb[3 * _B + pt]
                c100 = rwb[4 * _B + pt]
                c101 = rwb[5 * _B + pt]
                c110 = rwb[6 * _B + pt]
                c111 = rwb[7 * _B + pt]
                v00 = c000 + wx * (c001 - c000)
                v01 = c010 + wx * (c011 - c010)
                v10 = c100 + wx * (c101 - c100)
                v11 = c110 + wx * (c111 - c110)
                v0 = v00 + wy * (v01 - v00)
                v1 = v10 + wy * (v11 - v10)
                out_v[pt] = v0 + wz * (v1 - v0)

        def finish(b, par):
            drain(par)

            def phase_b(g, c2):
                combine_group(g, par)
                return c2

            lax.fori_loop(0, _B // _L, phase_b, 0)
            pltpu.sync_copy(out_v, out_hbm.at[pl.ds(obase + b * _B, _B)])

        prep(gb_base, 0)

        def pipe(bb, carry):
            for par in (0, 1):
                b = 2 * bb + par
                prep(gb_base + b + 1, 1 - par)
                finish(b, par)
            return carry

        lax.fori_loop(0, _NFB // 2, pipe, 0)
        drain(0)

        def tail(g, carry):
            combine_group(g, 0)
            pltpu.sync_copy(
                out_v.at[pl.ds(g * _L, _L)],
                out_hbm.at[pl.ds(obase + _FULL_PER_W + g * _L, _L)])
            return carry

        lax.fori_loop(0, tailg, tail, 0)

    return sampler


def _obase(w):
    return w * _FULL_PER_W + 32 * min(w, 24) + 48 * max(w - 24, 0)


def kernel(input, param):
    tab = input.transpose(1, 2, 3, 0).reshape(_V, _C)
    seg = _NBLK * _B
    pt = jnp.pad(param.transpose(1, 0),
                 ((0, 0), (0, _obase(_NW - 1) + seg - _N)))
    segs = jnp.stack(
        [lax.slice(pt, (0, _obase(w)), (3, _obase(w) + seg))
         for w in range(_NW)])
    crd = segs.reshape(_NW, 3, _NBLK, _B).transpose(0, 2, 1, 3).reshape(-1)
    return _sc_sampler()(tab, crd)


# speedup vs baseline: 1.0282x; 1.0282x over previous
import functools

import jax
import jax.numpy as jnp
from jax import lax
from jax.experimental import pallas as pl
from jax.experimental.pallas import tpu as pltpu
from jax.experimental.pallas import tpu_sc as plsc

_C = 16
_N = 2_000_000
_NW = 32
_L = 16
_B = 256
_NFB = 244
_NBLK = _NFB + 3
_V = 128 * 128 * 128
_FULL_PER_W = _NFB * _B


def _sc_sampler():
    mesh = plsc.VectorSubcoreMesh(core_axis_name="c", subcore_axis_name="s")

    @functools.partial(
        pl.kernel,
        mesh=mesh,
        compiler_params=pltpu.CompilerParams(use_tc_tiling_on_sc=False),
        out_type=jax.ShapeDtypeStruct((_N, _C), jnp.float32),
        scratch_types=[
            pltpu.VMEM((3 * _B,), jnp.float32),
            pltpu.VMEM((3 * _B,), jnp.float32),
            pltpu.VMEM((8 * _B,), jnp.int32),
            pltpu.VMEM((8 * _B,), jnp.int32),
            pltpu.VMEM((3 * _B,), jnp.float32),
            pltpu.VMEM((3 * _B,), jnp.float32),
            pltpu.VMEM((8 * _B, _C), jnp.float32),
            pltpu.VMEM((8 * _B, _C), jnp.float32),
            pltpu.VMEM((_B, _C), jnp.float32),
            pltpu.VMEM((_B, _C), jnp.float32),
            pltpu.SemaphoreType.DMA,
            pltpu.SemaphoreType.DMA,
            pltpu.SemaphoreType.DMA,
            pltpu.SemaphoreType.DMA,
            pltpu.SemaphoreType.DMA,
            pltpu.SemaphoreType.DMA,
        ],
    )
    def sampler(tab_hbm, crd_hbm, out_hbm, cb0, cb1, ix0, ix1, wb0, wb1,
                rw0, rw1, ob0, ob1, sg0, sg1, sc0, sc1, so0, so1):
        wid = lax.axis_index("s") * 2 + lax.axis_index("c")
        gb_base = wid * _NBLK
        obase = wid * _FULL_PER_W + 32 * jnp.minimum(wid, 24) \
            + 48 * jnp.maximum(wid - 24, 0)
        tailg = jnp.where(wid < 24, 2, 3)
        cbufs, ixs, wbs, rws, obs = (cb0, cb1), (ix0, ix1), (wb0, wb1), \
            (rw0, rw1), (ob0, ob1)
        sgs, scs, sos = (sg0, sg1), (sc0, sc1), (so0, so1)

        def fire_coords(b, par):
            gb = gb_base + b
            pltpu.async_copy(crd_hbm.at[pl.ds(gb * 3 * _B, 3 * _B)],
                             cbufs[par], scs[par])

        def prep(b, par):
            cbuf, ixb, wbf, rwb, sem = \
                cbufs[par], ixs[par], wbs[par], rws[par], sgs[par]
            pltpu.make_async_copy(crd_hbm.at[pl.ds(0, 3 * _B)], cbuf,
                                  scs[par]).wait()

            def phase_a(g, c2):
                s = g * _L
                fx = jnp.clip(cbuf[pl.ds(s, _L)] * 63.5 + 63.5, 0.0, 127.0)
                fy = jnp.clip(cbuf[pl.ds(_B + s, _L)] * 63.5 + 63.5,
                              0.0, 127.0)
                fz = jnp.clip(cbuf[pl.ds(2 * _B + s, _L)] * 63.5 + 63.5,
                              0.0, 127.0)
                x0 = fx.astype(jnp.int32)
                y0 = fy.astype(jnp.int32)
                z0 = fz.astype(jnp.int32)
                wx1 = fx - x0.astype(jnp.float32)
                wy1 = fy - y0.astype(jnp.float32)
                wz1 = fz - z0.astype(jnp.float32)
                x1 = jnp.minimum(x0 + 1, 127)
                ys0 = y0 << 7
                ys1 = jnp.minimum(y0 + 1, 127) << 7
                zs0 = z0 << 14
                zs1 = jnp.minimum(z0 + 1, 127) << 14
                b00 = zs0 + ys0
                b01 = zs0 + ys1
                b10 = zs1 + ys0
                b11 = zs1 + ys1
                bases = (b00, b00, b01, b01, b10, b10, b11, b11)
                xks = (x0, x1, x0, x1, x0, x1, x0, x1)
                for k in range(8):
                    ixb[pl.ds(k * _B + s, _L)] = bases[k] + xks[k]
                wbf[pl.ds(s, _L)] = wx1
                wbf[pl.ds(_B + s, _L)] = wy1
                wbf[pl.ds(2 * _B + s, _L)] = wz1
                return c2

            lax.fori_loop(0, _B // _L, phase_a, 0)
            for j in range(8 * (_B // 128)):
                jsl = pl.ds(j * 128, 128)
                pltpu.async_copy(tab_hbm.at[ixb.at[jsl]], rwb.at[jsl], sem)
            fire_coords(b + 2, par)

        def drain(par):
            pltpu.make_async_copy(tab_hbm.at[pl.ds(0, 8 * _B)], rws[par],
                                  sgs[par]).wait()

        def drain_out(par):
            pltpu.make_async_copy(tab_hbm.at[pl.ds(0, _B)], obs[par],
                                  sos[par]).wait()

        def drain_coords(par):
            pltpu.make_async_copy(crd_hbm.at[pl.ds(0, 3 * _B)], cbufs[par],
                                  scs[par]).wait()

        def combine_group(g, par):
            wbf, rwb, out_v = wbs[par], rws[par], obs[par]
            s = g * _L
            wxv = wbf[pl.ds(s, _L)]
            wyv = wbf[pl.ds(_B + s, _L)]
            wzv = wbf[pl.ds(2 * _B + s, _L)]
            for p in range(_L):
                pt = s + p
                wx, wy, wz = wxv[p], wyv[p], wzv[p]
                c000 = rwb[pt]
                c001 = rwb[_B + pt]
                c010 = rwb[2 * _B + pt]
                c011 = rwb[3 * _B + pt]
                c100 = rwb[4 * _B + pt]
                c101 = rwb[5 * _B + pt]
                c110 = rwb[6 * _B + pt]
                c111 = rwb[7 * _B + pt]
                v00 = c000 + wx * (c001 - c000)
                v01 = c010 + wx * (c011 - c010)
                v10 = c100 + wx * (c101 - c100)
                v11 = c110 + wx * (c111 - c110)
                v0 = v00 + wy * (v01 - v00)
                v1 = v10 + wy * (v11 - v10)
                out_v[pt] = v0 + wz * (v1 - v0)

        def finish(b, par):
            drain(par)

            @pl.when(b >= 2)
            def _():
                drain_out(par)

            def phase_b(g, c2):
                combine_group(g, par)
                return c2

            lax.fori_loop(0, _B // _L, phase_b, 0)
            pltpu.async_copy(obs[par], out_hbm.at[pl.ds(obase + b * _B, _B)],
                             sos[par])

        fire_coords(0, 0)
        fire_coords(1, 1)
        prep(0, 0)

        def pipe(bb, carry):
            for par in (0, 1):
                b = 2 * bb + par
                prep(b + 1, 1 - par)
                finish(b, par)
            return carry

        lax.fori_loop(0, _NFB // 2, pipe, 0)
        drain_coords(0)
        drain_coords(1)
        drain_out(0)
        drain_out(1)
        drain(0)

        def tail(g, carry):
            combine_group(g, 0)
            pltpu.sync_copy(
                obs[0].at[pl.ds(g * _L, _L)],
                out_hbm.at[pl.ds(obase + _FULL_PER_W + g * _L, _L)])
            return carry

        lax.fori_loop(0, tailg, tail, 0)

    return sampler


def _obase(w):
    return w * _FULL_PER_W + 32 * min(w, 24) + 48 * max(w - 24, 0)


def kernel(input, param):
    tab = input.transpose(1, 2, 3, 0).reshape(_V, _C)
    seg = _NBLK * _B
    pt = jnp.pad(param.transpose(1, 0),
                 ((0, 0), (0, _obase(_NW - 1) + seg - _N)))
    segs = jnp.stack(
        [lax.slice(pt, (0, _obase(w)), (3, _obase(w) + seg))
         for w in range(_NW)])
    crd = segs.reshape(_NW, 3, _NBLK, _B).transpose(0, 2, 1, 3).reshape(-1)
    return _sc_sampler()(tab, crd)
